# baseline (speedup 1.0000x reference)
import dataclasses
import functools

import jax
import jax.numpy as jnp
from jax import lax
from jax.experimental import pallas as pl
from jax.experimental.pallas import tpu as pltpu
from jax.experimental.pallas import tpu_sc as plsc

B = 16384
CTX = 20
PATH = 20
D = 32
NROW = 1000001
L = 16
NW = 32
GB = 4
NBLK = B // GB
BLK_PER_W = NBLK // NW
ROWS = GB * CTX
DEPTH = 2
RENB = 8192


def _tc_renorm(table):
    nflat = NROW * D
    npad = (RENB * D) - (nflat % (RENB * D))
    grid = (nflat + npad) // (RENB * D)

    def body(x_ref, o_ref):
        x = x_ref[...].reshape(RENB // 4, 4, D)
        n2 = jnp.sum(x * x, axis=2, keepdims=True)
        norms = jnp.sqrt(n2)
        scale = jnp.where(norms > 1.0, 1.0 / (norms + 1e-7), 1.0)
        o_ref[...] = (x * scale).reshape(RENB // 4, 4 * D)

    tflat = jnp.concatenate(
        [table.reshape(-1), jnp.zeros((npad,), jnp.float32)]).reshape(-1, 4 * D)
    out = pl.pallas_call(
        body,
        grid=(grid,),
        in_specs=[pl.BlockSpec((RENB // 4, 4 * D), lambda i: (i, 0))],
        out_specs=pl.BlockSpec((RENB // 4, 4 * D), lambda i: (i, 0)),
        out_shape=jax.ShapeDtypeStruct(tflat.shape, jnp.float32),
    )(tflat)
    return out.reshape(-1)[:nflat].reshape(NROW, D)


def _sc_excitations(t_renormed, ctx_idx, node_idx):
    mesh = plsc.VectorSubcoreMesh(core_axis_name="c", subcore_axis_name="s")
    cp = pltpu.CompilerParams()
    fields = pltpu.CompilerParams.__dataclass_fields__
    if "needs_layout_passes" in fields:
        cp = dataclasses.replace(cp, needs_layout_passes=False)
    if "use_tc_tiling_on_sc" in fields:
        cp = dataclasses.replace(cp, use_tc_tiling_on_sc=False)

    @functools.partial(
        pl.kernel,
        out_type=jax.ShapeDtypeStruct((NBLK, GB * D), jnp.float32),
        mesh=mesh,
        compiler_params=cp,
        scratch_types=[
            pltpu.VMEM((BLK_PER_W, ROWS), jnp.int32),
            pltpu.VMEM((BLK_PER_W, ROWS), jnp.int32),
            pltpu.VMEM((DEPTH, ROWS, D), jnp.float32),
            pltpu.VMEM((DEPTH, ROWS, D), jnp.float32),
            pltpu.VMEM((DEPTH, GB * D), jnp.float32),
        ] + [pltpu.SemaphoreType.DMA] * (2 * DEPTH),
    )
    def kern(t_hbm, ci_hbm, ni_hbm, out_hbm, ci_all, ni_all, cr, nr, ex,
             *sems):
        gsem = sems[:DEPTH]
        osem = sems[DEPTH:]
        wid = lax.axis_index("s") * 2 + lax.axis_index("c")
        base = wid * BLK_PER_W

        pltpu.sync_copy(ci_hbm.at[pl.ds(base, BLK_PER_W)], ci_all)
        pltpu.sync_copy(ni_hbm.at[pl.ds(base, BLK_PER_W)], ni_all)

        def start_gathers(b, p):
            pltpu.async_copy(t_hbm.at[ci_all.at[b]], cr.at[p], gsem[p])
            pltpu.async_copy(t_hbm.at[ni_all.at[b]], nr.at[p], gsem[p])

        def wait_gathers(b, p):
            pltpu.make_async_copy(t_hbm.at[ci_all.at[b]], cr.at[p],
                                  gsem[p]).wait()
            pltpu.make_async_copy(t_hbm.at[ni_all.at[b]], nr.at[p],
                                  gsem[p]).wait()

        for p in range(DEPTH):
            start_gathers(p, p)

        @pl.loop(0, BLK_PER_W, step=DEPTH)
        def _(j):
          for p in range(DEPTH):
            b = j + p
            wait_gathers(b, p)
            cr_v = cr.at[p]
            nr_v = nr.at[p]
            ex_v = ex.at[p]
            @pl.when(b >= DEPTH)
            def _():
                pltpu.make_async_copy(ex.at[p], out_hbm.at[base + b],
                                      osem[p]).wait()
            for bi in range(GB):
                cb0 = jnp.zeros((L,), jnp.float32)
                cb1 = jnp.zeros((L,), jnp.float32)
                for r in range(CTX):
                    row = bi * CTX + r
                    cb0 = cb0 + cr_v[row, pl.ds(0, L)]
                    cb1 = cb1 + cr_v[row, pl.ds(L, L)]
                lane = lax.iota(jnp.int32, L)
                e_lo = jnp.zeros((L,), jnp.float32)
                e_hi = jnp.zeros((L,), jnp.float32)
                for r in range(PATH):
                    row = bi * PATH + r
                    v0 = nr_v[row, pl.ds(0, L)]
                    v1 = nr_v[row, pl.ds(L, L)]
                    e = jnp.sum(v0 * cb0 + v1 * cb1)
                    if r < L:
                        e_lo = jnp.where(lane == r, e, e_lo)
                    else:
                        e_hi = jnp.where(lane == (r - L), e, e_hi)
                ex_v[pl.ds(bi * D, L)] = e_lo
                ex_v[pl.ds(bi * D + L, L)] = e_hi
            pltpu.async_copy(ex_v, out_hbm.at[base + b], osem[p])

            @pl.when(b + DEPTH < BLK_PER_W)
            def _():
                start_gathers(b + DEPTH, p)

        for p in range(DEPTH):
            pltpu.make_async_copy(ex.at[p], out_hbm.at[base], osem[p]).wait()

    return kern(t_renormed, ctx_idx, node_idx)


def _tc_loss(exc, turns, mask):
    def body(e_ref, t_ref, m_ref, o_ref):
        x = t_ref[...] * e_ref[:, :PATH]
        term = m_ref[...] / (1.0 + jnp.exp(-x))
        term = jnp.where(term == 0.0, 1.0, term)
        o_ref[0, 0] = -jnp.sum(jnp.log(term)) * (1.0 / B)

    return pl.pallas_call(
        body,
        out_shape=jax.ShapeDtypeStruct((1, 1), jnp.float32),
        out_specs=pl.BlockSpec(memory_space=pltpu.SMEM),
    )(exc, turns, mask)


def kernel(context, nodes, nodes_mask, turns_coeffs, table):
    ci = context.reshape(NBLK, ROWS)
    ni = nodes.reshape(NBLK, ROWS)
    exc = _sc_excitations(_tc_renorm(table), ci, ni)
    exc = exc.reshape(B, D)
    loss = _tc_loss(exc, turns_coeffs, nodes_mask)
    return loss[0, 0]


# speedup vs baseline: 3.5864x; 3.5864x over previous
import dataclasses
import functools

import jax
import jax.numpy as jnp
from jax import lax
from jax.experimental import pallas as pl
from jax.experimental.pallas import tpu as pltpu
from jax.experimental.pallas import tpu_sc as plsc

B = 16384
CTX = 20
PATH = 20
D = 32
NROW = 1000001
L = 16
NW = 32
GB = 4
NBLK = B // GB
BLK_PER_W = NBLK // NW
ROWS = GB * CTX
DEPTH = 2


def _rsqrt_nr(y_in):
    i = lax.bitcast_convert_type(y_in, jnp.int32)
    i = jnp.int32(0x5F3759DF) - lax.shift_right_logical(i, 1)
    y = lax.bitcast_convert_type(i, jnp.float32)
    for _ in range(3):
        y = y * (1.5 - 0.5 * y_in * y * y)
    return y


def _scales(n_lo, n_hi):
    s_lo = jnp.where(n_lo > 1.0, _rsqrt_nr(n_lo), 1.0)
    s_hi = jnp.where(n_hi > 1.0, _rsqrt_nr(n_hi), 1.0)
    return s_lo, s_hi


def _sc_excitations(table, ctx_idx, node_idx):
    mesh = plsc.VectorSubcoreMesh(core_axis_name="c", subcore_axis_name="s")
    cp = pltpu.CompilerParams()
    fields = pltpu.CompilerParams.__dataclass_fields__
    if "needs_layout_passes" in fields:
        cp = dataclasses.replace(cp, needs_layout_passes=False)
    if "use_tc_tiling_on_sc" in fields:
        cp = dataclasses.replace(cp, use_tc_tiling_on_sc=False)

    @functools.partial(
        pl.kernel,
        out_type=jax.ShapeDtypeStruct((NBLK, GB * D), jnp.float32),
        mesh=mesh,
        compiler_params=cp,
        scratch_types=[
            pltpu.VMEM((BLK_PER_W, ROWS), jnp.int32),
            pltpu.VMEM((BLK_PER_W, ROWS), jnp.int32),
            pltpu.VMEM((DEPTH, ROWS, D), jnp.float32),
            pltpu.VMEM((DEPTH, ROWS, D), jnp.float32),
            pltpu.VMEM((DEPTH, GB * D), jnp.float32),
            pltpu.VMEM((D,), jnp.float32),
        ] + [pltpu.SemaphoreType.DMA] * (2 * DEPTH),
    )
    def kern(t_hbm, ci_hbm, ni_hbm, out_hbm, ci_all, ni_all, cr, nr, ex,
             scl, *sems):
        gsem = sems[:DEPTH]
        osem = sems[DEPTH:]
        wid = lax.axis_index("s") * 2 + lax.axis_index("c")
        base = wid * BLK_PER_W

        pltpu.sync_copy(ci_hbm.at[pl.ds(base, BLK_PER_W)], ci_all)
        pltpu.sync_copy(ni_hbm.at[pl.ds(base, BLK_PER_W)], ni_all)

        def start_gathers(b, p):
            pltpu.async_copy(t_hbm.at[ci_all.at[b]], cr.at[p], gsem[p])
            pltpu.async_copy(t_hbm.at[ni_all.at[b]], nr.at[p], gsem[p])

        def wait_gathers(b, p):
            pltpu.make_async_copy(t_hbm.at[ci_all.at[b]], cr.at[p],
                                  gsem[p]).wait()
            pltpu.make_async_copy(t_hbm.at[ni_all.at[b]], nr.at[p],
                                  gsem[p]).wait()

        for p in range(DEPTH):
            start_gathers(p, p)

        @pl.loop(0, BLK_PER_W, step=DEPTH)
        def _(j):
          for p in range(DEPTH):
            b = j + p
            wait_gathers(b, p)
            cr_v = cr.at[p]
            nr_v = nr.at[p]
            ex_v = ex.at[p]
            @pl.when(b >= DEPTH)
            def _():
                pltpu.make_async_copy(ex.at[p], out_hbm.at[base + b],
                                      osem[p]).wait()
            lane = lax.iota(jnp.int32, L)
            for bi in range(GB):
                n_lo = jnp.zeros((L,), jnp.float32)
                n_hi = jnp.zeros((L,), jnp.float32)
                for r in range(CTX):
                    row = bi * CTX + r
                    v0 = cr_v[row, pl.ds(0, L)]
                    v1 = cr_v[row, pl.ds(L, L)]
                    n2 = jnp.sum(v0 * v0 + v1 * v1)
                    if r < L:
                        n_lo = jnp.where(lane == r, n2, n_lo)
                    else:
                        n_hi = jnp.where(lane == (r - L), n2, n_hi)
                s_lo, s_hi = _scales(n_lo, n_hi)
                scl[pl.ds(0, L)] = s_lo
                scl[pl.ds(L, L)] = s_hi
                cb0 = jnp.zeros((L,), jnp.float32)
                cb1 = jnp.zeros((L,), jnp.float32)
                for r in range(CTX):
                    row = bi * CTX + r
                    s = plsc.load_gather(scl, [jnp.full((L,), r, jnp.int32)])
                    cb0 = cb0 + cr_v[row, pl.ds(0, L)] * s
                    cb1 = cb1 + cr_v[row, pl.ds(L, L)] * s
                n_lo = jnp.zeros((L,), jnp.float32)
                n_hi = jnp.zeros((L,), jnp.float32)
                e_lo = jnp.zeros((L,), jnp.float32)
                e_hi = jnp.zeros((L,), jnp.float32)
                for r in range(PATH):
                    row = bi * PATH + r
                    v0 = nr_v[row, pl.ds(0, L)]
                    v1 = nr_v[row, pl.ds(L, L)]
                    n2 = jnp.sum(v0 * v0 + v1 * v1)
                    e = jnp.sum(v0 * cb0 + v1 * cb1)
                    if r < L:
                        n_lo = jnp.where(lane == r, n2, n_lo)
                        e_lo = jnp.where(lane == r, e, e_lo)
                    else:
                        n_hi = jnp.where(lane == (r - L), n2, n_hi)
                        e_hi = jnp.where(lane == (r - L), e, e_hi)
                s_lo, s_hi = _scales(n_lo, n_hi)
                ex_v[pl.ds(bi * D, L)] = e_lo * s_lo
                ex_v[pl.ds(bi * D + L, L)] = e_hi * s_hi
            pltpu.async_copy(ex_v, out_hbm.at[base + b], osem[p])

            @pl.when(b + DEPTH < BLK_PER_W)
            def _():
                start_gathers(b + DEPTH, p)

        for p in range(DEPTH):
            pltpu.make_async_copy(ex.at[p], out_hbm.at[base], osem[p]).wait()

    return kern(table, ctx_idx, node_idx)


def _tc_loss(exc, turns, mask):
    def body(e_ref, t_ref, m_ref, o_ref):
        x = t_ref[...] * e_ref[:, :PATH]
        term = m_ref[...] / (1.0 + jnp.exp(-x))
        term = jnp.where(term == 0.0, 1.0, term)
        o_ref[0, 0] = -jnp.sum(jnp.log(term)) * (1.0 / B)

    return pl.pallas_call(
        body,
        out_shape=jax.ShapeDtypeStruct((1, 1), jnp.float32),
        out_specs=pl.BlockSpec(memory_space=pltpu.SMEM),
    )(exc, turns, mask)


def kernel(context, nodes, nodes_mask, turns_coeffs, table):
    ci = context.reshape(NBLK, ROWS)
    ni = nodes.reshape(NBLK, ROWS)
    exc = _sc_excitations(table, ci, ni)
    exc = exc.reshape(B, D)
    loss = _tc_loss(exc, turns_coeffs, nodes_mask)
    return loss[0, 0]
